# Initial kernel scaffold; baseline (speedup 1.0000x reference)
import jax
import jax.numpy as jnp
from jax.experimental import pallas as pl


def kernel(x, targets, rpn_w, rpn_b, cls_w, cls_b, reg_w, reg_b):
    raise NotImplementedError("write your pallas kernel here")


# speedup vs baseline: 4.9964x; 4.9964x over previous
import functools

import jax
import jax.numpy as jnp
from jax.experimental import pallas as pl
from jax.experimental.pallas import tpu as pltpu

B = 2
FEATURE_DIM = 128
NUM_ANCHORS = 2
H = 100
W = 100
SCORE_T = 0.5
NMS_T = 0.7
STRIDE = 8.0
N_GT = 20
PRE_NMS_K = 1000
M = H * W * NUM_ANCHORS


def _anchors_t():
    ys = (jnp.arange(H, dtype=jnp.float32) + 0.5) * STRIDE
    xs = (jnp.arange(W, dtype=jnp.float32) + 0.5) * STRIDE
    gy, gx = jnp.meshgrid(ys, xs, indexing='ij')
    sizes = jnp.array([[16., 16., 4.], [32., 32., 6.]], dtype=jnp.float32)[:NUM_ANCHORS]
    cx = jnp.broadcast_to(gx[:, :, None], (H, W, NUM_ANCHORS))
    cy = jnp.broadcast_to(gy[:, :, None], (H, W, NUM_ANCHORS))
    cz = jnp.full((H, W, NUM_ANCHORS), 1.0, dtype=jnp.float32)
    sx = jnp.broadcast_to(sizes[None, None, :, 0], (H, W, NUM_ANCHORS))
    sy = jnp.broadcast_to(sizes[None, None, :, 1], (H, W, NUM_ANCHORS))
    sz = jnp.broadcast_to(sizes[None, None, :, 2], (H, W, NUM_ANCHORS))
    a = jnp.stack([cx, cy, cz, sx, sy, sz], axis=-1)
    a = jnp.concatenate([a, jnp.zeros((H, W, NUM_ANCHORS, 3), dtype=jnp.float32)], axis=-1)
    return a.reshape(-1, 9).T.reshape(1, 9, M)


def _post_body(obj_ref, reg_ref, anch_ref, tgt_ref,
               prop_ref, masked_ref, lobj_ref, lreg_ref):
    o = obj_ref[0]
    rT = reg_ref[0]
    aT = anch_ref[0]
    tT = tgt_ref[0]

    xyz = rT[0:3] * aT[3:6] + aT[0:3]
    size = aT[3:6] * jnp.exp(jnp.clip(rT[3:6], -10.0, 10.0))
    rot = aT[6:9] + rT[6:9]
    prop_ref[0] = jnp.concatenate([xyz, size, rot], axis=0)

    s = jax.nn.sigmoid(o)
    masked_ref[0] = jnp.where(s >= SCORE_T, s, -1.0)

    ax1 = aT[0:1] - aT[3:4] * 0.5
    ax2 = aT[0:1] + aT[3:4] * 0.5
    ay1 = aT[1:2] - aT[4:5] * 0.5
    ay2 = aT[1:2] + aT[4:5] * 0.5
    area_b = (ax2 - ax1) * (ay2 - ay1)

    best = jnp.full((1, M), -jnp.inf, dtype=jnp.float32)
    midx = jnp.zeros((1, M), dtype=jnp.int32)
    lq = jnp.zeros((1, M), dtype=jnp.bool_)
    for g in range(N_GT):
        gx = tT[0:1, g:g + 1]
        gy = tT[1:2, g:g + 1]
        gsx = tT[3:4, g:g + 1]
        gsy = tT[4:5, g:g + 1]
        gx1 = gx - gsx * 0.5
        gx2 = gx + gsx * 0.5
        gy1 = gy - gsy * 0.5
        gy2 = gy + gsy * 0.5
        iw = jnp.clip(jnp.minimum(gx2, ax2) - jnp.maximum(gx1, ax1), 0.0)
        ih = jnp.clip(jnp.minimum(gy2, ay2) - jnp.maximum(gy1, ay1), 0.0)
        inter = iw * ih
        area_a = (gx2 - gx1) * (gy2 - gy1)
        iou = inter / (area_a + area_b - inter + 1e-8)
        bpg = jnp.max(iou)
        lq = lq | (iou >= bpg - 1e-7)
        upd = iou > best
        best = jnp.where(upd, iou, best)
        midx = jnp.where(upd, g, midx)

    idxs = jnp.where(best < 0.2, -1, jnp.where(best < 0.6, -2, midx))
    idxs = jnp.where(lq, midx, idxs)
    gidx = jnp.maximum(idxs, 0)
    mg = jnp.zeros((9, M), dtype=jnp.float32)
    for g in range(N_GT):
        mg = jnp.where(gidx == g, tT[:, g:g + 1], mg)

    t1 = (mg[0:3] - aT[0:3]) / aT[3:6]
    t2 = jnp.log(jnp.maximum(mg[3:6], 1e-4) / aT[3:6])
    t3 = mg[6:9] - aT[6:9]
    rt = jnp.concatenate([t1, t2, t3], axis=0)
    d = rT - rt
    ad = jnp.abs(d)
    lreg = jnp.sum(jnp.where(ad < 1.0, 0.5 * d * d, ad - 0.5))

    lab = (idxs >= 0).astype(jnp.float32)
    lab = jnp.where(idxs == -1, 0.0, lab)
    lab = jnp.where(idxs == -2, -1.0, lab)
    lobj = jnp.sum(jnp.maximum(o, 0.0) - o * lab + jnp.log1p(jnp.exp(-jnp.abs(o))))

    lobj_ref[0] = jnp.full((1, 128), lobj, dtype=jnp.float32)
    lreg_ref[0] = jnp.full((1, 128), lreg, dtype=jnp.float32)


def _nms_body(tb_ref, tbT_ref, ts_ref, keep_ref, dscore_ref, dboxT_ref, iou_ref):
    tb = tb_ref[0]
    tbT = tbT_ref[0]
    ts = ts_ref[0]

    x1c = tb[:, 0:1] - tb[:, 3:4] * 0.5
    x2c = tb[:, 0:1] + tb[:, 3:4] * 0.5
    y1c = tb[:, 1:2] - tb[:, 4:5] * 0.5
    y2c = tb[:, 1:2] + tb[:, 4:5] * 0.5
    x1r = tbT[0:1] - tbT[3:4] * 0.5
    x2r = tbT[0:1] + tbT[3:4] * 0.5
    y1r = tbT[1:2] - tbT[4:5] * 0.5
    y2r = tbT[1:2] + tbT[4:5] * 0.5
    iw = jnp.clip(jnp.minimum(x2c, x2r) - jnp.maximum(x1c, x1r), 0.0)
    ih = jnp.clip(jnp.minimum(y2c, y2r) - jnp.maximum(y1c, y1r), 0.0)
    inter = iw * ih
    area_c = (x2c - x1c) * (y2c - y1c)
    area_r = (x2r - x1r) * (y2r - y1r)
    iou_ref[...] = inter / (area_c + area_r - inter + 1e-8)

    lanes = jax.lax.broadcasted_iota(jnp.int32, (1, PRE_NMS_K), 1)
    valid_f = jnp.where(ts >= SCORE_T, 1.0, 0.0)

    def body(i, keep_f):
        onehot = lanes == i
        k_i = jnp.max(jnp.where(onehot, keep_f, 0.0))
        row = iou_ref[pl.ds(i, 1), :]
        sup = (k_i > 0.0) & (row > NMS_T) & (lanes > i)
        return jnp.where(sup, 0.0, keep_f)

    keep_f = jax.lax.fori_loop(0, PRE_NMS_K, body, valid_f)
    keep = keep_f > 0.0
    keep_ref[0] = keep_f
    dscore_ref[0] = jnp.where(keep, ts, 0.0)
    dboxT_ref[0] = jnp.where(keep, tbT, 0.0)


@functools.partial(jax.jit, static_argnums=())
def kernel(x, targets, rpn_w, rpn_b, cls_w, cls_b, reg_w, reg_b):
    f = jax.nn.relu(jax.lax.conv_general_dilated(
        x, rpn_w, window_strides=(1, 1), padding='SAME',
        dimension_numbers=('NCHW', 'OIHW', 'NCHW')) + rpn_b[None, :, None, None])
    obj = jax.lax.conv_general_dilated(
        f, cls_w, window_strides=(1, 1), padding='VALID',
        dimension_numbers=('NCHW', 'OIHW', 'NCHW')) + cls_b[None, :, None, None]
    reg = jax.lax.conv_general_dilated(
        f, reg_w, window_strides=(1, 1), padding='VALID',
        dimension_numbers=('NCHW', 'OIHW', 'NCHW')) + reg_b[None, :, None, None]

    obj3 = jnp.transpose(obj, (0, 2, 3, 1)).reshape(B, 1, M)
    regT = jnp.transpose(reg.reshape(B, NUM_ANCHORS, 9, H, W),
                         (0, 2, 3, 4, 1)).reshape(B, 9, M)
    anchT = _anchors_t()
    tgtT = jnp.transpose(targets, (0, 2, 1))

    propT, masked, lobj, lreg = pl.pallas_call(
        _post_body,
        grid=(B,),
        in_specs=[
            pl.BlockSpec((1, 1, M), lambda b: (b, 0, 0)),
            pl.BlockSpec((1, 9, M), lambda b: (b, 0, 0)),
            pl.BlockSpec((1, 9, M), lambda b: (0, 0, 0)),
            pl.BlockSpec((1, 9, N_GT), lambda b: (b, 0, 0)),
        ],
        out_specs=[
            pl.BlockSpec((1, 9, M), lambda b: (b, 0, 0)),
            pl.BlockSpec((1, 1, M), lambda b: (b, 0, 0)),
            pl.BlockSpec((1, 1, 128), lambda b: (b, 0, 0)),
            pl.BlockSpec((1, 1, 128), lambda b: (b, 0, 0)),
        ],
        out_shape=[
            jax.ShapeDtypeStruct((B, 9, M), jnp.float32),
            jax.ShapeDtypeStruct((B, 1, M), jnp.float32),
            jax.ShapeDtypeStruct((B, 1, 128), jnp.float32),
            jax.ShapeDtypeStruct((B, 1, 128), jnp.float32),
        ],
    )(obj3, regT, anchT, tgtT)

    loss_obj = lobj[0, 0, 0] + lobj[1, 0, 0]
    loss_reg = lreg[0, 0, 0] + lreg[1, 0, 0]

    top_s, top_i = jax.lax.top_k(masked.reshape(B, M), PRE_NMS_K)
    tbT = jnp.take_along_axis(propT, top_i[:, None, :], axis=2)
    tb = jnp.transpose(tbT, (0, 2, 1))
    ts3 = top_s.reshape(B, 1, PRE_NMS_K)

    keep_f, dscore, dboxT = pl.pallas_call(
        _nms_body,
        grid=(B,),
        in_specs=[
            pl.BlockSpec((1, PRE_NMS_K, 9), lambda b: (b, 0, 0)),
            pl.BlockSpec((1, 9, PRE_NMS_K), lambda b: (b, 0, 0)),
            pl.BlockSpec((1, 1, PRE_NMS_K), lambda b: (b, 0, 0)),
        ],
        out_specs=[
            pl.BlockSpec((1, 1, PRE_NMS_K), lambda b: (b, 0, 0)),
            pl.BlockSpec((1, 1, PRE_NMS_K), lambda b: (b, 0, 0)),
            pl.BlockSpec((1, 9, PRE_NMS_K), lambda b: (b, 0, 0)),
        ],
        out_shape=[
            jax.ShapeDtypeStruct((B, 1, PRE_NMS_K), jnp.float32),
            jax.ShapeDtypeStruct((B, 1, PRE_NMS_K), jnp.float32),
            jax.ShapeDtypeStruct((B, 9, PRE_NMS_K), jnp.float32),
        ],
        scratch_shapes=[pltpu.VMEM((PRE_NMS_K, PRE_NMS_K), jnp.float32)],
    )(tb, tbT, ts3)

    det_boxes = jnp.transpose(dboxT, (0, 2, 1))
    det_scores = dscore.reshape(B, PRE_NMS_K)
    det_keep = keep_f.reshape(B, PRE_NMS_K) > 0.5
    return (det_boxes, det_scores, det_keep, loss_obj, loss_reg)


# speedup vs baseline: 5.2916x; 1.0591x over previous
import functools

import jax
import jax.numpy as jnp
from jax.experimental import pallas as pl
from jax.experimental.pallas import tpu as pltpu

B = 2
FEATURE_DIM = 128
NUM_ANCHORS = 2
H = 100
W = 100
SCORE_T = 0.5
NMS_T = 0.7
STRIDE = 8.0
N_GT = 20
PRE_NMS_K = 1000
M = H * W * NUM_ANCHORS


def _anchors_t():
    ys = (jnp.arange(H, dtype=jnp.float32) + 0.5) * STRIDE
    xs = (jnp.arange(W, dtype=jnp.float32) + 0.5) * STRIDE
    gy, gx = jnp.meshgrid(ys, xs, indexing='ij')
    sizes = jnp.array([[16., 16., 4.], [32., 32., 6.]], dtype=jnp.float32)[:NUM_ANCHORS]
    cx = jnp.broadcast_to(gx[:, :, None], (H, W, NUM_ANCHORS))
    cy = jnp.broadcast_to(gy[:, :, None], (H, W, NUM_ANCHORS))
    cz = jnp.full((H, W, NUM_ANCHORS), 1.0, dtype=jnp.float32)
    sx = jnp.broadcast_to(sizes[None, None, :, 0], (H, W, NUM_ANCHORS))
    sy = jnp.broadcast_to(sizes[None, None, :, 1], (H, W, NUM_ANCHORS))
    sz = jnp.broadcast_to(sizes[None, None, :, 2], (H, W, NUM_ANCHORS))
    a = jnp.stack([cx, cy, cz, sx, sy, sz], axis=-1)
    a = jnp.concatenate([a, jnp.zeros((H, W, NUM_ANCHORS, 3), dtype=jnp.float32)], axis=-1)
    return a.reshape(-1, 9).T.reshape(1, 9, M)


def _post_body(obj_ref, reg_ref, anch_ref, tgt_ref,
               prop_ref, masked_ref, lobj_ref, lreg_ref):
    o = obj_ref[0]
    rT = reg_ref[0]
    aT = anch_ref[0]
    tT = tgt_ref[0]

    xyz = rT[0:3] * aT[3:6] + aT[0:3]
    size = aT[3:6] * jnp.exp(jnp.clip(rT[3:6], -10.0, 10.0))
    rot = aT[6:9] + rT[6:9]
    prop_ref[0] = jnp.concatenate([xyz, size, rot], axis=0)

    s = jax.nn.sigmoid(o)
    masked_ref[0] = jnp.where(s >= SCORE_T, s, -1.0)

    ax1 = aT[0:1] - aT[3:4] * 0.5
    ax2 = aT[0:1] + aT[3:4] * 0.5
    ay1 = aT[1:2] - aT[4:5] * 0.5
    ay2 = aT[1:2] + aT[4:5] * 0.5
    area_b = (ax2 - ax1) * (ay2 - ay1)

    best = jnp.full((1, M), -jnp.inf, dtype=jnp.float32)
    midx = jnp.zeros((1, M), dtype=jnp.int32)
    lq = jnp.zeros((1, M), dtype=jnp.bool_)
    for g in range(N_GT):
        gx = tT[0:1, g:g + 1]
        gy = tT[1:2, g:g + 1]
        gsx = tT[3:4, g:g + 1]
        gsy = tT[4:5, g:g + 1]
        gx1 = gx - gsx * 0.5
        gx2 = gx + gsx * 0.5
        gy1 = gy - gsy * 0.5
        gy2 = gy + gsy * 0.5
        iw = jnp.clip(jnp.minimum(gx2, ax2) - jnp.maximum(gx1, ax1), 0.0)
        ih = jnp.clip(jnp.minimum(gy2, ay2) - jnp.maximum(gy1, ay1), 0.0)
        inter = iw * ih
        area_a = (gx2 - gx1) * (gy2 - gy1)
        iou = inter / (area_a + area_b - inter + 1e-8)
        bpg = jnp.max(iou)
        lq = lq | (iou >= bpg - 1e-7)
        upd = iou > best
        best = jnp.where(upd, iou, best)
        midx = jnp.where(upd, g, midx)

    idxs = jnp.where(best < 0.2, -1, jnp.where(best < 0.6, -2, midx))
    idxs = jnp.where(lq, midx, idxs)
    gidx = jnp.maximum(idxs, 0)
    mg = jnp.zeros((9, M), dtype=jnp.float32)
    for g in range(N_GT):
        mg = jnp.where(gidx == g, tT[:, g:g + 1], mg)

    t1 = (mg[0:3] - aT[0:3]) / aT[3:6]
    t2 = jnp.log(jnp.maximum(mg[3:6], 1e-4) / aT[3:6])
    t3 = mg[6:9] - aT[6:9]
    rt = jnp.concatenate([t1, t2, t3], axis=0)
    d = rT - rt
    ad = jnp.abs(d)
    lreg = jnp.sum(jnp.where(ad < 1.0, 0.5 * d * d, ad - 0.5))

    lab = (idxs >= 0).astype(jnp.float32)
    lab = jnp.where(idxs == -1, 0.0, lab)
    lab = jnp.where(idxs == -2, -1.0, lab)
    lobj = jnp.sum(jnp.maximum(o, 0.0) - o * lab + jnp.log1p(jnp.exp(-jnp.abs(o))))

    lobj_ref[0] = jnp.full((1, 128), lobj, dtype=jnp.float32)
    lreg_ref[0] = jnp.full((1, 128), lreg, dtype=jnp.float32)


def _nms_body(tb_ref, tbT_ref, ts_ref, keep_ref, dscore_ref, dboxT_ref, iou_ref):
    tb = tb_ref[0]
    tbT = tbT_ref[0]
    ts = ts_ref[0]

    x1c = tb[:, 0:1] - tb[:, 3:4] * 0.5
    x2c = tb[:, 0:1] + tb[:, 3:4] * 0.5
    y1c = tb[:, 1:2] - tb[:, 4:5] * 0.5
    y2c = tb[:, 1:2] + tb[:, 4:5] * 0.5
    x1r = tbT[0:1] - tbT[3:4] * 0.5
    x2r = tbT[0:1] + tbT[3:4] * 0.5
    y1r = tbT[1:2] - tbT[4:5] * 0.5
    y2r = tbT[1:2] + tbT[4:5] * 0.5
    iw = jnp.clip(jnp.minimum(x2c, x2r) - jnp.maximum(x1c, x1r), 0.0)
    ih = jnp.clip(jnp.minimum(y2c, y2r) - jnp.maximum(y1c, y1r), 0.0)
    inter = iw * ih
    area_c = (x2c - x1c) * (y2c - y1c)
    area_r = (x2r - x1r) * (y2r - y1r)
    iou_ref[...] = inter / (area_c + area_r - inter + 1e-8)

    lanes = jax.lax.broadcasted_iota(jnp.int32, (1, PRE_NMS_K), 1)
    valid_f = jnp.where(ts >= SCORE_T, 1.0, 0.0)

    S = 8
    lanes8 = jax.lax.broadcasted_iota(jnp.int32, (S, PRE_NMS_K), 1)
    rowi8 = jax.lax.broadcasted_iota(jnp.int32, (S, 1), 0)
    li8 = jax.lax.broadcasted_iota(jnp.int32, (1, S), 1)

    def blk_body(blk, keep_f):
        base = blk * S
        rows = iou_ref[pl.ds(base, S), :]
        sel = lanes8 == (base + rowi8)
        kcol = jnp.max(jnp.where(sel, keep_f, 0.0), axis=1, keepdims=True)
        kband = jnp.transpose(kcol)
        bblk = tb_ref[0, pl.ds(base, S), :]
        bT = jnp.transpose(bblk)
        bx1c = bblk[:, 0:1] - bblk[:, 3:4] * 0.5
        bx2c = bblk[:, 0:1] + bblk[:, 3:4] * 0.5
        by1c = bblk[:, 1:2] - bblk[:, 4:5] * 0.5
        by2c = bblk[:, 1:2] + bblk[:, 4:5] * 0.5
        bx1r = bT[0:1] - bT[3:4] * 0.5
        bx2r = bT[0:1] + bT[3:4] * 0.5
        by1r = bT[1:2] - bT[4:5] * 0.5
        by2r = bT[1:2] + bT[4:5] * 0.5
        biw = jnp.clip(jnp.minimum(bx2c, bx2r) - jnp.maximum(bx1c, bx1r), 0.0)
        bih = jnp.clip(jnp.minimum(by2c, by2r) - jnp.maximum(by1c, by1r), 0.0)
        binter = biw * bih
        bac = (bx2c - bx1c) * (by2c - by1c)
        bar = (bx2r - bx1r) * (by2r - by1r)
        dblk = binter / (bac + bar - binter + 1e-8)
        for j in range(S):
            kj = kband[0:1, j:j + 1]
            dj = dblk[j:j + 1, :]
            supj = (kj > 0.0) & (dj > NMS_T) & (li8 > j)
            kband = jnp.where(supj, 0.0, kband)
        kfin = jnp.transpose(kband)
        sup = (kfin > 0.0) & (rows > NMS_T) & (lanes8 > base + rowi8)
        sup_any = jnp.max(jnp.where(sup, 1.0, 0.0), axis=0, keepdims=True)
        return jnp.where(sup_any > 0.0, 0.0, keep_f)

    keep_f = jax.lax.fori_loop(0, PRE_NMS_K // S, blk_body, valid_f)
    keep = keep_f > 0.0
    keep_ref[0] = keep_f
    dscore_ref[0] = jnp.where(keep, ts, 0.0)
    dboxT_ref[0] = jnp.where(keep, tbT, 0.0)


@functools.partial(jax.jit, static_argnums=())
def kernel(x, targets, rpn_w, rpn_b, cls_w, cls_b, reg_w, reg_b):
    f = jax.nn.relu(jax.lax.conv_general_dilated(
        x, rpn_w, window_strides=(1, 1), padding='SAME',
        dimension_numbers=('NCHW', 'OIHW', 'NCHW')) + rpn_b[None, :, None, None])
    obj = jax.lax.conv_general_dilated(
        f, cls_w, window_strides=(1, 1), padding='VALID',
        dimension_numbers=('NCHW', 'OIHW', 'NCHW')) + cls_b[None, :, None, None]
    reg = jax.lax.conv_general_dilated(
        f, reg_w, window_strides=(1, 1), padding='VALID',
        dimension_numbers=('NCHW', 'OIHW', 'NCHW')) + reg_b[None, :, None, None]

    obj3 = jnp.transpose(obj, (0, 2, 3, 1)).reshape(B, 1, M)
    regT = jnp.transpose(reg.reshape(B, NUM_ANCHORS, 9, H, W),
                         (0, 2, 3, 4, 1)).reshape(B, 9, M)
    anchT = _anchors_t()
    tgtT = jnp.transpose(targets, (0, 2, 1))

    propT, masked, lobj, lreg = pl.pallas_call(
        _post_body,
        grid=(B,),
        in_specs=[
            pl.BlockSpec((1, 1, M), lambda b: (b, 0, 0)),
            pl.BlockSpec((1, 9, M), lambda b: (b, 0, 0)),
            pl.BlockSpec((1, 9, M), lambda b: (0, 0, 0)),
            pl.BlockSpec((1, 9, N_GT), lambda b: (b, 0, 0)),
        ],
        out_specs=[
            pl.BlockSpec((1, 9, M), lambda b: (b, 0, 0)),
            pl.BlockSpec((1, 1, M), lambda b: (b, 0, 0)),
            pl.BlockSpec((1, 1, 128), lambda b: (b, 0, 0)),
            pl.BlockSpec((1, 1, 128), lambda b: (b, 0, 0)),
        ],
        out_shape=[
            jax.ShapeDtypeStruct((B, 9, M), jnp.float32),
            jax.ShapeDtypeStruct((B, 1, M), jnp.float32),
            jax.ShapeDtypeStruct((B, 1, 128), jnp.float32),
            jax.ShapeDtypeStruct((B, 1, 128), jnp.float32),
        ],
    )(obj3, regT, anchT, tgtT)

    loss_obj = lobj[0, 0, 0] + lobj[1, 0, 0]
    loss_reg = lreg[0, 0, 0] + lreg[1, 0, 0]

    top_s, top_i = jax.lax.top_k(masked.reshape(B, M), PRE_NMS_K)
    tbT = jnp.take_along_axis(propT, top_i[:, None, :], axis=2)
    tb = jnp.transpose(tbT, (0, 2, 1))
    ts3 = top_s.reshape(B, 1, PRE_NMS_K)

    keep_f, dscore, dboxT = pl.pallas_call(
        _nms_body,
        grid=(B,),
        in_specs=[
            pl.BlockSpec((1, PRE_NMS_K, 9), lambda b: (b, 0, 0)),
            pl.BlockSpec((1, 9, PRE_NMS_K), lambda b: (b, 0, 0)),
            pl.BlockSpec((1, 1, PRE_NMS_K), lambda b: (b, 0, 0)),
        ],
        out_specs=[
            pl.BlockSpec((1, 1, PRE_NMS_K), lambda b: (b, 0, 0)),
            pl.BlockSpec((1, 1, PRE_NMS_K), lambda b: (b, 0, 0)),
            pl.BlockSpec((1, 9, PRE_NMS_K), lambda b: (b, 0, 0)),
        ],
        out_shape=[
            jax.ShapeDtypeStruct((B, 1, PRE_NMS_K), jnp.float32),
            jax.ShapeDtypeStruct((B, 1, PRE_NMS_K), jnp.float32),
            jax.ShapeDtypeStruct((B, 9, PRE_NMS_K), jnp.float32),
        ],
        scratch_shapes=[pltpu.VMEM((PRE_NMS_K, PRE_NMS_K), jnp.float32)],
    )(tb, tbT, ts3)

    det_boxes = jnp.transpose(dboxT, (0, 2, 1))
    det_scores = dscore.reshape(B, PRE_NMS_K)
    det_keep = keep_f.reshape(B, PRE_NMS_K) > 0.5
    return (det_boxes, det_scores, det_keep, loss_obj, loss_reg)


# speedup vs baseline: 6.7893x; 1.2830x over previous
import functools

import jax
import jax.numpy as jnp
from jax.experimental import pallas as pl
from jax.experimental.pallas import tpu as pltpu

B = 2
FEATURE_DIM = 128
NUM_ANCHORS = 2
H = 100
W = 100
SCORE_T = 0.5
NMS_T = 0.7
STRIDE = 8.0
N_GT = 20
PRE_NMS_K = 1000
M = H * W * NUM_ANCHORS


def _anchors_t():
    ys = (jnp.arange(H, dtype=jnp.float32) + 0.5) * STRIDE
    xs = (jnp.arange(W, dtype=jnp.float32) + 0.5) * STRIDE
    gy, gx = jnp.meshgrid(ys, xs, indexing='ij')
    sizes = jnp.array([[16., 16., 4.], [32., 32., 6.]], dtype=jnp.float32)[:NUM_ANCHORS]
    cx = jnp.broadcast_to(gx[:, :, None], (H, W, NUM_ANCHORS))
    cy = jnp.broadcast_to(gy[:, :, None], (H, W, NUM_ANCHORS))
    cz = jnp.full((H, W, NUM_ANCHORS), 1.0, dtype=jnp.float32)
    sx = jnp.broadcast_to(sizes[None, None, :, 0], (H, W, NUM_ANCHORS))
    sy = jnp.broadcast_to(sizes[None, None, :, 1], (H, W, NUM_ANCHORS))
    sz = jnp.broadcast_to(sizes[None, None, :, 2], (H, W, NUM_ANCHORS))
    a = jnp.stack([cx, cy, cz, sx, sy, sz], axis=-1)
    a = jnp.concatenate([a, jnp.zeros((H, W, NUM_ANCHORS, 3), dtype=jnp.float32)], axis=-1)
    return a.reshape(-1, 9).T.reshape(1, 9, M)


def _post_body(obj_ref, reg_ref, anch_ref, tgt_ref,
               prop_ref, masked_ref, lobj_ref, lreg_ref):
    o = obj_ref[0]
    rT = reg_ref[0]
    aT = anch_ref[0]
    tT = tgt_ref[0]

    xyz = rT[0:3] * aT[3:6] + aT[0:3]
    size = aT[3:6] * jnp.exp(jnp.clip(rT[3:6], -10.0, 10.0))
    rot = aT[6:9] + rT[6:9]
    prop_ref[0] = jnp.concatenate([xyz, size, rot], axis=0)

    s = jax.nn.sigmoid(o)
    masked_ref[0] = jnp.where(s >= SCORE_T, s, -1.0)

    ax1 = aT[0:1] - aT[3:4] * 0.5
    ax2 = aT[0:1] + aT[3:4] * 0.5
    ay1 = aT[1:2] - aT[4:5] * 0.5
    ay2 = aT[1:2] + aT[4:5] * 0.5
    area_b = (ax2 - ax1) * (ay2 - ay1)

    best = jnp.full((1, M), -jnp.inf, dtype=jnp.float32)
    midx = jnp.zeros((1, M), dtype=jnp.int32)
    lq = jnp.zeros((1, M), dtype=jnp.bool_)
    for g in range(N_GT):
        gx = tT[0:1, g:g + 1]
        gy = tT[1:2, g:g + 1]
        gsx = tT[3:4, g:g + 1]
        gsy = tT[4:5, g:g + 1]
        gx1 = gx - gsx * 0.5
        gx2 = gx + gsx * 0.5
        gy1 = gy - gsy * 0.5
        gy2 = gy + gsy * 0.5
        iw = jnp.clip(jnp.minimum(gx2, ax2) - jnp.maximum(gx1, ax1), 0.0)
        ih = jnp.clip(jnp.minimum(gy2, ay2) - jnp.maximum(gy1, ay1), 0.0)
        inter = iw * ih
        area_a = (gx2 - gx1) * (gy2 - gy1)
        iou = inter / (area_a + area_b - inter + 1e-8)
        bpg = jnp.max(iou)
        lq = lq | (iou >= bpg - 1e-7)
        upd = iou > best
        best = jnp.where(upd, iou, best)
        midx = jnp.where(upd, g, midx)

    idxs = jnp.where(best < 0.2, -1, jnp.where(best < 0.6, -2, midx))
    idxs = jnp.where(lq, midx, idxs)
    gidx = jnp.maximum(idxs, 0)
    mg = jnp.zeros((9, M), dtype=jnp.float32)
    for g in range(N_GT):
        mg = jnp.where(gidx == g, tT[:, g:g + 1], mg)

    t1 = (mg[0:3] - aT[0:3]) / aT[3:6]
    t2 = jnp.log(jnp.maximum(mg[3:6], 1e-4) / aT[3:6])
    t3 = mg[6:9] - aT[6:9]
    rt = jnp.concatenate([t1, t2, t3], axis=0)
    d = rT - rt
    ad = jnp.abs(d)
    lreg = jnp.sum(jnp.where(ad < 1.0, 0.5 * d * d, ad - 0.5))

    lab = (idxs >= 0).astype(jnp.float32)
    lab = jnp.where(idxs == -1, 0.0, lab)
    lab = jnp.where(idxs == -2, -1.0, lab)
    lobj = jnp.sum(jnp.maximum(o, 0.0) - o * lab + jnp.log1p(jnp.exp(-jnp.abs(o))))

    lobj_ref[0] = jnp.full((1, 128), lobj, dtype=jnp.float32)
    lreg_ref[0] = jnp.full((1, 128), lreg, dtype=jnp.float32)


def _nms_body(tb_ref, tbT_ref, ts_ref, keep_ref, dscore_ref, dboxT_ref, iou_ref):
    tb = tb_ref[0]
    tbT = tbT_ref[0]
    ts = ts_ref[0]

    x1c = tb[:, 0:1] - tb[:, 3:4] * 0.5
    x2c = tb[:, 0:1] + tb[:, 3:4] * 0.5
    y1c = tb[:, 1:2] - tb[:, 4:5] * 0.5
    y2c = tb[:, 1:2] + tb[:, 4:5] * 0.5
    x1r = tbT[0:1] - tbT[3:4] * 0.5
    x2r = tbT[0:1] + tbT[3:4] * 0.5
    y1r = tbT[1:2] - tbT[4:5] * 0.5
    y2r = tbT[1:2] + tbT[4:5] * 0.5
    iw = jnp.clip(jnp.minimum(x2c, x2r) - jnp.maximum(x1c, x1r), 0.0)
    ih = jnp.clip(jnp.minimum(y2c, y2r) - jnp.maximum(y1c, y1r), 0.0)
    inter = iw * ih
    area_c = (x2c - x1c) * (y2c - y1c)
    area_r = (x2r - x1r) * (y2r - y1r)
    iou_ref[...] = inter / (area_c + area_r - inter + 1e-8)

    lanes = jax.lax.broadcasted_iota(jnp.int32, (1, PRE_NMS_K), 1)
    valid_f = jnp.where(ts >= SCORE_T, 1.0, 0.0)

    S = 8
    lanes8 = jax.lax.broadcasted_iota(jnp.int32, (S, PRE_NMS_K), 1)
    rowi8 = jax.lax.broadcasted_iota(jnp.int32, (S, 1), 0)
    li8 = jax.lax.broadcasted_iota(jnp.int32, (1, S), 1)

    def blk_body(blk, keep_f):
        base = blk * S
        rows = iou_ref[pl.ds(base, S), :]
        sel = lanes8 == (base + rowi8)
        kcol = jnp.max(jnp.where(sel, keep_f, 0.0), axis=1, keepdims=True)
        bblk = tb_ref[0, pl.ds(base, S), :]
        bT = jnp.transpose(bblk)
        bx1c = bblk[:, 0:1] - bblk[:, 3:4] * 0.5
        bx2c = bblk[:, 0:1] + bblk[:, 3:4] * 0.5
        by1c = bblk[:, 1:2] - bblk[:, 4:5] * 0.5
        by2c = bblk[:, 1:2] + bblk[:, 4:5] * 0.5
        bx1r = bT[0:1] - bT[3:4] * 0.5
        bx2r = bT[0:1] + bT[3:4] * 0.5
        by1r = bT[1:2] - bT[4:5] * 0.5
        by2r = bT[1:2] + bT[4:5] * 0.5
        biw = jnp.clip(jnp.minimum(bx2c, bx2r) - jnp.maximum(bx1c, bx1r), 0.0)
        bih = jnp.clip(jnp.minimum(by2c, by2r) - jnp.maximum(by1c, by1r), 0.0)
        binter = biw * bih
        bac = (bx2c - bx1c) * (by2c - by1c)
        bar = (bx2r - bx1r) * (by2r - by1r)
        dblk = binter / (bac + bar - binter + 1e-8)
        for j in range(S):
            kj = kcol[j:j + 1, 0:1]
            dj = dblk[:, j:j + 1]
            supj = (kj > 0.0) & (dj > NMS_T) & (rowi8 > j)
            kcol = jnp.where(supj, 0.0, kcol)
        sup = (kcol > 0.0) & (rows > NMS_T) & (lanes8 > base + rowi8)
        sup_any = jnp.max(jnp.where(sup, 1.0, 0.0), axis=0, keepdims=True)
        return jnp.where(sup_any > 0.0, 0.0, keep_f)

    keep_f = jax.lax.fori_loop(0, PRE_NMS_K // S, blk_body, valid_f)
    keep = keep_f > 0.0
    keep_ref[0] = keep_f
    dscore_ref[0] = jnp.where(keep, ts, 0.0)
    dboxT_ref[0] = jnp.where(keep, tbT, 0.0)


@functools.partial(jax.jit, static_argnums=())
def kernel(x, targets, rpn_w, rpn_b, cls_w, cls_b, reg_w, reg_b):
    f = jax.nn.relu(jax.lax.conv_general_dilated(
        x, rpn_w, window_strides=(1, 1), padding='SAME',
        dimension_numbers=('NCHW', 'OIHW', 'NCHW')) + rpn_b[None, :, None, None])
    obj = jax.lax.conv_general_dilated(
        f, cls_w, window_strides=(1, 1), padding='VALID',
        dimension_numbers=('NCHW', 'OIHW', 'NCHW')) + cls_b[None, :, None, None]
    reg = jax.lax.conv_general_dilated(
        f, reg_w, window_strides=(1, 1), padding='VALID',
        dimension_numbers=('NCHW', 'OIHW', 'NCHW')) + reg_b[None, :, None, None]

    obj3 = jnp.transpose(obj, (0, 2, 3, 1)).reshape(B, 1, M)
    regT = jnp.transpose(reg.reshape(B, NUM_ANCHORS, 9, H, W),
                         (0, 2, 3, 4, 1)).reshape(B, 9, M)
    anchT = _anchors_t()
    tgtT = jnp.transpose(targets, (0, 2, 1))

    propT, masked, lobj, lreg = pl.pallas_call(
        _post_body,
        grid=(B,),
        in_specs=[
            pl.BlockSpec((1, 1, M), lambda b: (b, 0, 0)),
            pl.BlockSpec((1, 9, M), lambda b: (b, 0, 0)),
            pl.BlockSpec((1, 9, M), lambda b: (0, 0, 0)),
            pl.BlockSpec((1, 9, N_GT), lambda b: (b, 0, 0)),
        ],
        out_specs=[
            pl.BlockSpec((1, 9, M), lambda b: (b, 0, 0)),
            pl.BlockSpec((1, 1, M), lambda b: (b, 0, 0)),
            pl.BlockSpec((1, 1, 128), lambda b: (b, 0, 0)),
            pl.BlockSpec((1, 1, 128), lambda b: (b, 0, 0)),
        ],
        out_shape=[
            jax.ShapeDtypeStruct((B, 9, M), jnp.float32),
            jax.ShapeDtypeStruct((B, 1, M), jnp.float32),
            jax.ShapeDtypeStruct((B, 1, 128), jnp.float32),
            jax.ShapeDtypeStruct((B, 1, 128), jnp.float32),
        ],
    )(obj3, regT, anchT, tgtT)

    loss_obj = lobj[0, 0, 0] + lobj[1, 0, 0]
    loss_reg = lreg[0, 0, 0] + lreg[1, 0, 0]

    top_s, top_i = jax.lax.top_k(masked.reshape(B, M), PRE_NMS_K)
    tbT = jnp.take_along_axis(propT, top_i[:, None, :], axis=2)
    tb = jnp.transpose(tbT, (0, 2, 1))
    ts3 = top_s.reshape(B, 1, PRE_NMS_K)

    keep_f, dscore, dboxT = pl.pallas_call(
        _nms_body,
        grid=(B,),
        in_specs=[
            pl.BlockSpec((1, PRE_NMS_K, 9), lambda b: (b, 0, 0)),
            pl.BlockSpec((1, 9, PRE_NMS_K), lambda b: (b, 0, 0)),
            pl.BlockSpec((1, 1, PRE_NMS_K), lambda b: (b, 0, 0)),
        ],
        out_specs=[
            pl.BlockSpec((1, 1, PRE_NMS_K), lambda b: (b, 0, 0)),
            pl.BlockSpec((1, 1, PRE_NMS_K), lambda b: (b, 0, 0)),
            pl.BlockSpec((1, 9, PRE_NMS_K), lambda b: (b, 0, 0)),
        ],
        out_shape=[
            jax.ShapeDtypeStruct((B, 1, PRE_NMS_K), jnp.float32),
            jax.ShapeDtypeStruct((B, 1, PRE_NMS_K), jnp.float32),
            jax.ShapeDtypeStruct((B, 9, PRE_NMS_K), jnp.float32),
        ],
        scratch_shapes=[pltpu.VMEM((PRE_NMS_K, PRE_NMS_K), jnp.float32)],
    )(tb, tbT, ts3)

    det_boxes = jnp.transpose(dboxT, (0, 2, 1))
    det_scores = dscore.reshape(B, PRE_NMS_K)
    det_keep = keep_f.reshape(B, PRE_NMS_K) > 0.5
    return (det_boxes, det_scores, det_keep, loss_obj, loss_reg)
